# Initial kernel scaffold; baseline (speedup 1.0000x reference)
import jax
import jax.numpy as jnp
from jax.experimental import pallas as pl


def kernel(X, params, edge_index, batch):
    raise NotImplementedError("write your pallas kernel here")


# speedup vs baseline: 5.2539x; 5.2539x over previous
import functools

import jax
import jax.numpy as jnp
from jax import lax
from jax.experimental import pallas as pl
from jax.experimental.pallas import tpu as pltpu
from jax.experimental.pallas import tpu_sc as plsc

N = 10000
F_IN = 128
H = 256
C = 16
G = 64
EPS_BN = 1e-5

BLK = 1024
NBLK = 10
NPAD = BLK * NBLK
HH = 128
NHALF = NPAD // 2
NTILES = 16
RPT = NHALF // NTILES


def _layer_a_body(y_ref, st_ref, w_ref, a2t_ref, a2_ref,
                  h2_ref, al_ref, acc0_ref):
    x = y_ref[...] * st_ref[0:1, :] + st_ref[1:2, :]
    h = jnp.dot(x, w_ref[...], preferred_element_type=jnp.float32)
    al = lax.dot_general(a2_ref[...], h, (((1,), (1,)), ((), ())),
                         preferred_element_type=jnp.float32)
    asd_row = al[0:1, :] + al[1:2, :]
    wself_row = jnp.exp(jnp.where(asd_row >= 0, asd_row, asd_row * 0.2))
    al_ref[...] = al
    al_ref[0:1, :] = jnp.exp(al[0:1, :])
    al_ref[1:2, :] = jnp.exp(al[1:2, :])
    al_ref[2:3, :] = wself_row
    al_ref[3:4, :] = jnp.exp(al[0:1, :] * 0.2)
    al_ref[4:5, :] = jnp.exp(al[1:2, :] * 0.2)
    alc = jnp.dot(h, a2t_ref[...], preferred_element_type=jnp.float32)
    asd = alc[:, 0:1] + alc[:, 1:2]
    wself = jnp.exp(jnp.where(asd >= 0, asd, asd * 0.2))
    for p in range(2):
        hp = h[:, p * HH:(p + 1) * HH]
        h2_ref[p] = hp
        acc0_ref[0, p] = hp * wself


def _layer_a(y, st, W, A2T, A2):
    din = y.shape[1]
    return pl.pallas_call(
        _layer_a_body,
        grid=(NBLK,),
        in_specs=[
            pl.BlockSpec((BLK, din), lambda nb: (nb, 0)),
            pl.BlockSpec((8, din), lambda nb: (0, 0)),
            pl.BlockSpec((din, H), lambda nb: (0, 0)),
            pl.BlockSpec((H, 16), lambda nb: (0, 0)),
            pl.BlockSpec((16, H), lambda nb: (0, 0)),
        ],
        out_specs=[
            pl.BlockSpec((2, BLK, HH), lambda nb: (0, nb, 0)),
            pl.BlockSpec((16, BLK), lambda nb: (0, nb)),
            pl.BlockSpec((1, 2, BLK, HH), lambda nb: (nb // 5, 0, nb % 5, 0)),
        ],
        out_shape=[
            jax.ShapeDtypeStruct((2, NPAD, HH), jnp.float32),
            jax.ShapeDtypeStruct((16, NPAD), jnp.float32),
            jax.ShapeDtypeStruct((2, 2, NHALF, HH), jnp.float32),
        ],
    )(y, st, W, A2T, A2)


def _sc_edge_body(ept, h2, al, srcv, dstv, acc0, acc_out, den_out,
                  src_r, dst_r, asrc_r, adst_r, qsrc_r, qdst_r,
                  rows_r, stage_r, idx_r, den_r, acc_sh, sem):
    cid = lax.axis_index("c")
    sid = lax.axis_index("s")
    pltpu.sync_copy(al.at[0], asrc_r)
    pltpu.sync_copy(al.at[1], adst_r)
    pltpu.sync_copy(al.at[3], qsrc_r)
    pltpu.sync_copy(al.at[4], qdst_r)
    pltpu.sync_copy(al.at[5, pl.ds(0, NHALF)], den_r)

    m0 = lax.iota(jnp.int32, 16) == 0
    lo = cid * NHALF

    def vgather(v, idx):
        return lax.gather(
            v, idx.reshape(16, 1),
            lax.GatherDimensionNumbers(offset_dims=(),
                                       collapsed_slice_dims=(0,),
                                       start_index_map=(0,)),
            (1,), mode=lax.GatherScatterMode.PROMISE_IN_BOUNDS)

    def bcast(v, j):
        return vgather(v, jnp.full((16,), j, jnp.int32))

    iota16 = lax.iota(jnp.int32, 16)

    for p in range(2):
        pbase = p * NPAD
        plsc.subcore_barrier()
        pltpu.sync_copy(acc0.at[cid, p, pl.ds(sid * RPT, RPT)],
                        acc_sh.at[pl.ds(sid * RPT, RPT)])
        plsc.subcore_barrier()

        def chunk(g, carry):
            b = pl.multiple_of(g * 16, 16)
            src16 = src_r[pl.ds(b, 16)]
            dst16 = dst_r[pl.ds(b, 16)]
            dstl = dst16 - lo
            owned = (dstl >= 0) & (dstl < NHALF)
            dstl = jnp.where(owned, dstl, 0)
            p_s = plsc.load_gather(asrc_r, [src16])
            p_d = plsc.load_gather(adst_r, [dst16])
            q_s = plsc.load_gather(qsrc_r, [src16])
            q_d = plsc.load_gather(qdst_r, [dst16])
            e1 = p_s * p_d
            w = jnp.where(e1 >= 1.0, e1, q_s * q_d)
            w = jnp.where(owned, w, 0.0)
            if p == 0:
                sk, sv = plsc.sort_key_val(dstl, w)
                csum = plsc.cumsum(sv)
                prev = vgather(sk, jnp.maximum(iota16 - 1, 0))
                is_start = (iota16 == 0) | (sk != prev)
                nxt = vgather(sk, jnp.minimum(iota16 + 1, 15))
                is_end = (iota16 == 15) | (sk != nxt)
                marks = jnp.where(is_start, iota16, -1)
                start_idx = plsc.cummax(marks)
                ex = csum - sv
                pe = vgather(ex, start_idx)
                total = csum - pe
                plsc.addupdate_scatter(den_r, [sk], total, mask=is_end)
            idx_r[...] = src16 + pbase
            pltpu.async_copy(h2.at[idx_r], rows_r, sem).wait()
            for j in range(16):
                wj = bcast(w, j)
                for k in range(HH // 16):
                    stage_r[j, pl.ds(k * 16, 16)] = (
                        rows_r[j, pl.ds(k * 16, 16)] * wj)
            pltpu.sync_copy(stage_r, acc_sh.at[dstl], add=True)
            return carry

        for win in range(2):
            ew = ept // 2
            pltpu.sync_copy(srcv.at[pl.ds(sid * ept + win * ew, ew)], src_r)
            pltpu.sync_copy(dstv.at[pl.ds(sid * ept + win * ew, ew)], dst_r)
            lax.fori_loop(0, ew // 16, chunk, 0)
        plsc.subcore_barrier()
        pltpu.sync_copy(acc_sh.at[pl.ds(sid * RPT, RPT)],
                        acc_out.at[cid, p, pl.ds(sid * RPT, RPT)])

    pltpu.sync_copy(den_r, den_out.at[cid, sid])


def _sc_edge(h2flat, al, src, dst, acc0):
    e_tot = src.shape[0]
    ept = e_tot // NTILES
    mesh = plsc.VectorSubcoreMesh(core_axis_name="c", subcore_axis_name="s")
    kfn = pl.kernel(
        functools.partial(_sc_edge_body, ept),
        out_type=[
            jax.ShapeDtypeStruct((2, 2, NHALF, HH), jnp.float32),
            jax.ShapeDtypeStruct((2, NTILES, NHALF), jnp.float32),
        ],
        mesh=mesh,
        scratch_types=[
            pltpu.VMEM((ept // 2,), jnp.int32),
            pltpu.VMEM((ept // 2,), jnp.int32),
            pltpu.VMEM((NPAD,), jnp.float32),
            pltpu.VMEM((NPAD,), jnp.float32),
            pltpu.VMEM((NPAD,), jnp.float32),
            pltpu.VMEM((NPAD,), jnp.float32),
            pltpu.VMEM((16, HH), jnp.float32),
            pltpu.VMEM((16, HH), jnp.float32),
            pltpu.VMEM((16,), jnp.int32),
            pltpu.VMEM((NHALF,), jnp.float32),
            pltpu.VMEM_SHARED((NHALF, HH), jnp.float32),
            pltpu.SemaphoreType.DMA,
        ],
        compiler_params=pltpu.CompilerParams(needs_layout_passes=False),
    )
    return kfn(h2flat, al, src, dst, acc0)


def _layer_b_body(acc_ref, dg_ref, al_ref, b_ref, g_ref, be_ref,
                  y_ref, st_ref):
    nb = pl.program_id(0)
    ones16 = jnp.ones((NTILES, 1), jnp.float32)
    e2 = (lax.broadcasted_iota(jnp.int32, (16, 1), 0) == 2).astype(jnp.float32)
    den = lax.dot_general(dg_ref[0], ones16, (((0,), (0,)), ((), ())),
                          precision=lax.Precision.HIGHEST,
                          preferred_element_type=jnp.float32)
    den += lax.dot_general(al_ref[...], e2, (((0,), (0,)), ((), ())),
                           precision=lax.Precision.HIGHEST,
                           preferred_element_type=jnp.float32)
    den += 1e-16
    a = acc_ref[0]
    hc = jnp.concatenate([a[0], a[1]], axis=1)
    y = jnp.maximum(hc / den + b_ref[0:1, :], 0.0)
    y_ref[...] = y

    @pl.when(nb == 0)
    def _init():
        st_ref[...] = jnp.zeros((8, H), jnp.float32)

    rid = nb * BLK + lax.broadcasted_iota(jnp.int32, (BLK, 1), 0)
    ym = jnp.where(rid < N, y, 0.0)
    st_ref[0:1, :] += jnp.sum(ym, axis=0, keepdims=True)
    st_ref[1:2, :] += jnp.sum(ym * ym, axis=0, keepdims=True)

    @pl.when(nb == NBLK - 1)
    def _fold():
        mu = st_ref[0:1, :] / N
        var = st_ref[1:2, :] / N - mu * mu
        s = g_ref[0:1, :] * lax.rsqrt(var + EPS_BN)
        st_ref[0:1, :] = s
        st_ref[1:2, :] = be_ref[0:1, :] - mu * s


def _layer_b(acc, dg, al, b2, g2, be2):
    return pl.pallas_call(
        _layer_b_body,
        grid=(NBLK,),
        in_specs=[
            pl.BlockSpec((1, 2, BLK, HH), lambda nb: (nb // 5, 0, nb % 5, 0)),
            pl.BlockSpec((1, NTILES, BLK), lambda nb: (nb // 5, 0, nb % 5)),
            pl.BlockSpec((16, BLK), lambda nb: (0, nb)),
            pl.BlockSpec((1, H), lambda nb: (0, 0)),
            pl.BlockSpec((1, H), lambda nb: (0, 0)),
            pl.BlockSpec((1, H), lambda nb: (0, 0)),
        ],
        out_specs=[
            pl.BlockSpec((BLK, H), lambda nb: (nb, 0)),
            pl.BlockSpec((8, H), lambda nb: (0, 0)),
        ],
        out_shape=[
            jax.ShapeDtypeStruct((NPAD, H), jnp.float32),
            jax.ShapeDtypeStruct((8, H), jnp.float32),
        ],
        compiler_params=pltpu.CompilerParams(
            dimension_semantics=("arbitrary",)),
    )(acc, dg, al, b2, g2, be2)


def _layer_c_body(y_ref, st_ref, batch_ref, wl_ref, bl_ref,
                  xout_ref, pooled_ref, psum_ref, csum_ref):
    nb = pl.program_id(0)

    @pl.when(nb == 0)
    def _init():
        psum_ref[...] = jnp.zeros((G, H), jnp.float32)
        csum_ref[...] = jnp.zeros((G, 128), jnp.float32)

    gids = lax.broadcasted_iota(jnp.int32, (G, BLK), 0)
    oh = (batch_ref[0] == gids).astype(jnp.float32)
    psum_ref[...] += jnp.dot(oh, y_ref[...],
                             precision=lax.Precision.HIGHEST,
                             preferred_element_type=jnp.float32)
    csum_ref[...] += jnp.broadcast_to(
        jnp.sum(oh, axis=1, keepdims=True), (G, 128))

    @pl.when(nb == NBLK - 1)
    def _fin():
        cnt = csum_ref[:, 0:1]
        cntm = jnp.maximum(cnt, 1.0)
        s = st_ref[0:1, :]
        t = st_ref[1:2, :]
        pooled = jnp.where(cnt > 0, psum_ref[...] * s / cntm + t, 0.0)
        pooled_ref[...] = pooled
        logits = jnp.dot(pooled, wl_ref[...],
                         preferred_element_type=jnp.float32) + bl_ref[0:1, :]
        mx = jnp.max(logits, axis=1, keepdims=True)
        ex = jnp.exp(logits - mx)
        xout_ref[...] = ex / jnp.sum(ex, axis=1, keepdims=True)


def _layer_c(y, st, batch3d, Wlin, blin2):
    return pl.pallas_call(
        _layer_c_body,
        grid=(NBLK,),
        in_specs=[
            pl.BlockSpec((BLK, H), lambda nb: (nb, 0)),
            pl.BlockSpec((8, H), lambda nb: (0, 0)),
            pl.BlockSpec((1, 1, BLK), lambda nb: (nb, 0, 0)),
            pl.BlockSpec((H, C), lambda nb: (0, 0)),
            pl.BlockSpec((1, C), lambda nb: (0, 0)),
        ],
        out_specs=[
            pl.BlockSpec((G, C), lambda nb: (0, 0)),
            pl.BlockSpec((G, H), lambda nb: (0, 0)),
        ],
        out_shape=[
            jax.ShapeDtypeStruct((G, C), jnp.float32),
            jax.ShapeDtypeStruct((G, H), jnp.float32),
        ],
        scratch_shapes=[
            pltpu.VMEM((G, H), jnp.float32),
            pltpu.VMEM((G, 128), jnp.float32),
        ],
        compiler_params=pltpu.CompilerParams(
            dimension_semantics=("arbitrary",)),
    )(y, st, batch3d, Wlin, blin2)


def kernel(X, params, edge_index, batch):
    src = edge_index[0]
    dst = edge_index[1]
    Xp = jnp.pad(X, ((0, NPAD - N), (0, 0)))
    batch3d = jnp.pad(batch, (0, NPAD - N),
                      constant_values=G).reshape(NBLK, 1, BLK)

    y = Xp
    st = jnp.concatenate(
        [jnp.ones((1, F_IN), jnp.float32),
         jnp.zeros((7, F_IN), jnp.float32)], axis=0)
    for i in range(5):
        a_src = params[f"asrc{i}"]
        a_dst = params[f"adst{i}"]
        A2 = jnp.zeros((16, H), jnp.float32)
        A2 = A2.at[0].set(a_src).at[1].set(a_dst)
        A2T = jnp.zeros((H, 16), jnp.float32)
        A2T = A2T.at[:, 0].set(a_src).at[:, 1].set(a_dst)
        h2, al, acc0 = _layer_a(y, st, params[f"W{i}"], A2T, A2)
        acc, dg = _sc_edge(h2.reshape(2 * NPAD, HH), al, src, dst, acc0)
        dg2 = dg
        y, st = _layer_b(acc, dg2, al,
                         params[f"b{i}"].reshape(1, H),
                         params[f"gamma{i}"].reshape(1, H),
                         params[f"beta{i}"].reshape(1, H))
    x_out, pooled = _layer_c(y, st, batch3d,
                             params["Wlin"], params["blin"].reshape(1, C))
    return (x_out, pooled)
